# Initial kernel scaffold; baseline (speedup 1.0000x reference)
import jax
import jax.numpy as jnp
from jax.experimental import pallas as pl


def kernel(x, U, thres_mean, thres_var):
    raise NotImplementedError("write your pallas kernel here")


# speedup vs baseline: 1.8912x; 1.8912x over previous
import numpy as np
import jax
import jax.numpy as jnp
from jax.experimental import pallas as pl
from jax.experimental.pallas import tpu as pltpu

_N = 4
_D1 = 64
_D2 = 16
_D = _D1 * _D2
_B = 32768
_NCODES = 8
_NUM_PART = 5
_RES_DENOS = (float(2**2 - 1), float(2**2 + 1), float(2**4 + 1))


def _ste_floor(x):
    return x + (jnp.floor(x) - x)


def _quant_kernel(u_ref, thres_ref, w_ref):
    U = u_ref[...]
    thres = thres_ref[...]
    beta = jnp.max(U)
    alpha = jnp.min(U)

    lane = jax.lax.broadcasted_iota(jnp.int32, (1, _D), 1)

    s = (beta - alpha) / _RES_DENOS[0]
    vals_sum = s * _ste_floor(U / s)

    for lvl in (1, 2):
        s = s / _RES_DENOS[lvl]
        res = U - vals_sum
        rows = [res[i:i + 1, :] for i in range(_N)]

        rank = []
        for i in range(_N):
            acc = None
            for j in range(_N):
                if j == i:
                    continue
                c = (rows[j] <= rows[i]) if j < i else (rows[j] < rows[i])
                ci = c.astype(jnp.int32)
                acc = ci if acc is None else acc + ci
            rank.append(acc)

        sorted_err = []
        for k in range(_N):
            acc = None
            for i in range(_N):
                t = jnp.where(rank[i] == k, rows[i], 0.0)
                acc = t if acc is None else acc + t
            sorted_err.append(acc)

        delta = [sorted_err[k + 1] - sorted_err[k] for k in range(_N - 1)]
        dmax = jnp.max(jnp.maximum(jnp.maximum(delta[0], delta[1]), delta[2]))
        delta = [d / dmax for d in delta]

        msp = [jax.nn.sigmoid((d - thres) / 0.01) for d in delta]
        rs = [(d > thres).astype(jnp.int32) for d in delta]

        code = rs[0] * 4 + rs[1] * 2 + rs[2]

        counts = [jnp.sum((code == c).astype(jnp.int32)) for c in range(_NCODES)]

        pos = []
        for i in range(_NCODES):
            acc = jnp.int32(0)
            for j in range(_NCODES):
                if j == i:
                    continue
                if j < i:
                    acc = acc + (counts[j] >= counts[i]).astype(jnp.int32)
                else:
                    acc = acc + (counts[j] > counts[i]).astype(jnp.int32)
            pos.append(acc)
        order = []
        for p in range(_NCODES):
            acc = jnp.int32(0)
            for i in range(_NCODES):
                acc = acc + jnp.where(pos[i] == p, jnp.int32(i), jnp.int32(0))
            order.append(acc)
        top = order[:_NUM_PART]

        clustered = code
        for j in range(_NCODES - _NUM_PART):
            less = order[_NUM_PART + j]
            lb = [(less // 4) % 2, (less // 2) % 2, less % 2]
            best_val = None
            winner = None
            for t in range(_NUM_PART):
                tb = [(top[t] // 4) % 2, (top[t] // 2) % 2, top[t] % 2]
                inter = lb[0] * tb[0] + lb[1] * tb[1] + lb[2] * tb[2]
                if best_val is None:
                    best_val, winner = inter, top[t]
                else:
                    better = inter > best_val
                    best_val = jnp.where(better, inter, best_val)
                    winner = jnp.where(better, top[t], winner)
            clustered = jnp.where(clustered == less, winner, clustered)

        running = jnp.int32(0)
        uidx = []
        for c in range(_NCODES):
            running = running + (counts[c] > 0).astype(jnp.int32)
            uidx.append(running - 1)
        sidx = None
        for c in range(_NCODES):
            t = jnp.where(clustered == c, uidx[c], jnp.int32(0))
            sidx = t if sidx is None else sidx + t
        sidx = jnp.clip(sidx, 0, _NCODES - 1)

        sel = None
        m = [None] * (_N - 1)
        for p in range(_NCODES):
            col_mask = lane == p
            code_p = jnp.sum(jnp.where(col_mask, code, jnp.int32(0)))
            hit = sidx == p
            t = jnp.where(hit, code_p, jnp.int32(0))
            sel = t if sel is None else sel + t
            for k in range(_N - 1):
                msp_p = jnp.sum(jnp.where(col_mask, msp[k], 0.0))
                tv = jnp.where(hit, msp_p, 0.0)
                m[k] = tv if m[k] is None else m[k] + tv

        t_rows = []
        for pattern in range(_NCODES):
            bits = [(pattern >> (_N - 2 - i)) & 1 for i in range(_N - 1)]
            buf = jnp.zeros((1, _D), jnp.float32)
            cnt = 0
            grad_ = jnp.ones((1, _D), jnp.float32)
            rows_out = []
            for i in range(_N):
                buf = buf + sorted_err[i]
                cnt += 1
                if i == _N - 1:
                    v = grad_ * (buf / cnt)
                    rows_out.extend([v] * cnt)
                elif bits[i]:
                    grad_ = grad_ * m[i]
                    v = grad_ * (buf / cnt)
                    rows_out.extend([v] * cnt)
                    buf = jnp.zeros((1, _D), jnp.float32)
                    cnt = 0
                    grad_ = jnp.ones((1, _D), jnp.float32)
                else:
                    grad_ = grad_ * (1.0 - m[i])
            t_rows.append(rows_out)

        inner = []
        for c in range(_N):
            acc = None
            for pattern in range(_NCODES):
                t = jnp.where(sel == pattern, t_rows[pattern][c], 0.0)
                acc = t if acc is None else acc + t
            inner.append(acc)
        new_rows = []
        for i in range(_N):
            acc = None
            for c in range(_N):
                t = jnp.where(rank[i] == c, inner[c], 0.0)
                acc = t if acc is None else acc + t
            new_rows.append(acc)
        grouped = jnp.concatenate(new_rows, axis=0)

        vals_sum = vals_sum + s * _ste_floor(grouped / s)

    w_ref[...] = vals_sum


def _matmul_kernel(x_ref, w_ref, o_ref):
    o_ref[...] = jnp.dot(x_ref[...], w_ref[...],
                         preferred_element_type=jnp.float32)


_BT = 2048


def kernel(x, U, thres_mean, thres_var):
    del thres_var
    thres = jax.nn.sigmoid(jnp.repeat(thres_mean, _D1)).reshape(1, _D)

    w = pl.pallas_call(
        _quant_kernel,
        out_shape=jax.ShapeDtypeStruct((_N, _D), jnp.float32),
    )(U, thres)

    wq = w.reshape(_N, _D1, _D2)
    wblk = jnp.zeros((_N * _D1, _N * _D2), jnp.float32)
    for n in range(_N):
        wblk = wblk.at[n * _D1:(n + 1) * _D1, n * _D2:(n + 1) * _D2].set(wq[n])

    x2 = x.reshape(_B, _N * _D1)
    out = pl.pallas_call(
        _matmul_kernel,
        grid=(_B // _BT,),
        in_specs=[
            pl.BlockSpec((_BT, _N * _D1), lambda i: (i, 0)),
            pl.BlockSpec((_N * _D1, _N * _D2), lambda i: (0, 0)),
        ],
        out_specs=pl.BlockSpec((_BT, _N * _D2), lambda i: (i, 0)),
        out_shape=jax.ShapeDtypeStruct((_B, _N * _D2), jnp.float32),
    )(x2, wblk)
    return out.reshape(_B, _N, _D2)


# speedup vs baseline: 2.0053x; 1.0603x over previous
import numpy as np
import jax
import jax.numpy as jnp
from jax.experimental import pallas as pl
from jax.experimental.pallas import tpu as pltpu

_N = 4
_D1 = 64
_D2 = 16
_D = _D1 * _D2
_B = 32768
_NCODES = 8
_NUM_PART = 5
_RES_DENOS = (float(2**2 - 1), float(2**2 + 1), float(2**4 + 1))


def _ste_floor(x):
    return x + (jnp.floor(x) - x)


def _quant_kernel(u_ref, thres_ref, w_ref):
    U = u_ref[...]
    thres = thres_ref[...]
    beta = jnp.max(U)
    alpha = jnp.min(U)

    lane = jax.lax.broadcasted_iota(jnp.int32, (1, _D), 1)

    s = (beta - alpha) / _RES_DENOS[0]
    vals_sum = s * _ste_floor(U / s)

    for lvl in (1, 2):
        s = s / _RES_DENOS[lvl]
        res = U - vals_sum
        rows = [res[i:i + 1, :] for i in range(_N)]

        rank = []
        for i in range(_N):
            acc = None
            for j in range(_N):
                if j == i:
                    continue
                c = (rows[j] <= rows[i]) if j < i else (rows[j] < rows[i])
                ci = c.astype(jnp.int32)
                acc = ci if acc is None else acc + ci
            rank.append(acc)

        sorted_err = []
        for k in range(_N):
            acc = None
            for i in range(_N):
                t = jnp.where(rank[i] == k, rows[i], 0.0)
                acc = t if acc is None else acc + t
            sorted_err.append(acc)

        delta = [sorted_err[k + 1] - sorted_err[k] for k in range(_N - 1)]
        dmax = jnp.max(jnp.maximum(jnp.maximum(delta[0], delta[1]), delta[2]))
        delta = [d / dmax for d in delta]

        msp = [jax.nn.sigmoid((d - thres) / 0.01) for d in delta]
        rs = [(d > thres).astype(jnp.int32) for d in delta]

        code = rs[0] * 4 + rs[1] * 2 + rs[2]

        counts = [jnp.sum((code == c).astype(jnp.int32)) for c in range(_NCODES)]

        pos = []
        for i in range(_NCODES):
            acc = jnp.int32(0)
            for j in range(_NCODES):
                if j == i:
                    continue
                if j < i:
                    acc = acc + (counts[j] >= counts[i]).astype(jnp.int32)
                else:
                    acc = acc + (counts[j] > counts[i]).astype(jnp.int32)
            pos.append(acc)
        order = []
        for p in range(_NCODES):
            acc = jnp.int32(0)
            for i in range(_NCODES):
                acc = acc + jnp.where(pos[i] == p, jnp.int32(i), jnp.int32(0))
            order.append(acc)
        top = order[:_NUM_PART]

        clustered = code
        for j in range(_NCODES - _NUM_PART):
            less = order[_NUM_PART + j]
            lb = [(less // 4) % 2, (less // 2) % 2, less % 2]
            best_val = None
            winner = None
            for t in range(_NUM_PART):
                tb = [(top[t] // 4) % 2, (top[t] // 2) % 2, top[t] % 2]
                inter = lb[0] * tb[0] + lb[1] * tb[1] + lb[2] * tb[2]
                if best_val is None:
                    best_val, winner = inter, top[t]
                else:
                    better = inter > best_val
                    best_val = jnp.where(better, inter, best_val)
                    winner = jnp.where(better, top[t], winner)
            clustered = jnp.where(clustered == less, winner, clustered)

        running = jnp.int32(0)
        uidx = []
        for c in range(_NCODES):
            running = running + (counts[c] > 0).astype(jnp.int32)
            uidx.append(running - 1)
        sidx = None
        for c in range(_NCODES):
            t = jnp.where(clustered == c, uidx[c], jnp.int32(0))
            sidx = t if sidx is None else sidx + t
        sidx = jnp.clip(sidx, 0, _NCODES - 1)

        sel = None
        m = [None] * (_N - 1)
        for p in range(_NCODES):
            col_mask = lane == p
            code_p = jnp.sum(jnp.where(col_mask, code, jnp.int32(0)))
            hit = sidx == p
            t = jnp.where(hit, code_p, jnp.int32(0))
            sel = t if sel is None else sel + t
            for k in range(_N - 1):
                msp_p = jnp.sum(jnp.where(col_mask, msp[k], 0.0))
                tv = jnp.where(hit, msp_p, 0.0)
                m[k] = tv if m[k] is None else m[k] + tv

        t_rows = []
        for pattern in range(_NCODES):
            bits = [(pattern >> (_N - 2 - i)) & 1 for i in range(_N - 1)]
            buf = jnp.zeros((1, _D), jnp.float32)
            cnt = 0
            grad_ = jnp.ones((1, _D), jnp.float32)
            rows_out = []
            for i in range(_N):
                buf = buf + sorted_err[i]
                cnt += 1
                if i == _N - 1:
                    v = grad_ * (buf / cnt)
                    rows_out.extend([v] * cnt)
                elif bits[i]:
                    grad_ = grad_ * m[i]
                    v = grad_ * (buf / cnt)
                    rows_out.extend([v] * cnt)
                    buf = jnp.zeros((1, _D), jnp.float32)
                    cnt = 0
                    grad_ = jnp.ones((1, _D), jnp.float32)
                else:
                    grad_ = grad_ * (1.0 - m[i])
            t_rows.append(rows_out)

        inner = []
        for c in range(_N):
            acc = None
            for pattern in range(_NCODES):
                t = jnp.where(sel == pattern, t_rows[pattern][c], 0.0)
                acc = t if acc is None else acc + t
            inner.append(acc)
        new_rows = []
        for i in range(_N):
            acc = None
            for c in range(_N):
                t = jnp.where(rank[i] == c, inner[c], 0.0)
                acc = t if acc is None else acc + t
            new_rows.append(acc)
        grouped = jnp.concatenate(new_rows, axis=0)

        vals_sum = vals_sum + s * _ste_floor(grouped / s)

    w_ref[...] = vals_sum


def _matmul_kernel(x_ref, w_ref, o_ref):
    o_ref[...] = jnp.dot(x_ref[...], w_ref[...],
                         preferred_element_type=jnp.float32)


_BT = 4096


def kernel(x, U, thres_mean, thres_var):
    del thres_var
    thres = jax.nn.sigmoid(jnp.repeat(thres_mean, _D1)).reshape(1, _D)

    w = pl.pallas_call(
        _quant_kernel,
        out_shape=jax.ShapeDtypeStruct((_N, _D), jnp.float32),
    )(U, thres)

    wq = w.reshape(_N, _D1, _D2)
    wblk = jnp.zeros((_N * _D1, _N * _D2), jnp.float32)
    for n in range(_N):
        wblk = wblk.at[n * _D1:(n + 1) * _D1, n * _D2:(n + 1) * _D2].set(wq[n])

    x2 = x.reshape(_B, _N * _D1)
    out = pl.pallas_call(
        _matmul_kernel,
        grid=(_B // _BT,),
        in_specs=[
            pl.BlockSpec((_BT, _N * _D1), lambda i: (i, 0)),
            pl.BlockSpec((_N * _D1, _N * _D2), lambda i: (0, 0)),
        ],
        out_specs=pl.BlockSpec((_BT, _N * _D2), lambda i: (i, 0)),
        out_shape=jax.ShapeDtypeStruct((_B, _N * _D2), jnp.float32),
    )(x2, wblk)
    return out.reshape(_B, _N, _D2)


# speedup vs baseline: 2.1805x; 1.0874x over previous
import jax
import jax.numpy as jnp
from jax.experimental import pallas as pl
from jax.experimental.pallas import tpu as pltpu

_N = 4
_D1 = 64
_D2 = 16
_D = _D1 * _D2
_B = 32768
_NCODES = 8
_NUM_PART = 5
_RES_DENOS = (float(2**2 - 1), float(2**2 + 1), float(2**4 + 1))
_BT = 4096


def _ste_floor(x):
    return x + (jnp.floor(x) - x)


def _quantize(U_rows, thres):
    allU = jnp.concatenate(U_rows, axis=0)
    beta = jnp.max(allU)
    alpha = jnp.min(allU)

    sub = jax.lax.broadcasted_iota(jnp.int32, (_D1, _D2), 0)
    lane = jax.lax.broadcasted_iota(jnp.int32, (_D1, _D2), 1)

    s = (beta - alpha) / _RES_DENOS[0]
    vals = [s * _ste_floor(u / s) for u in U_rows]

    for lvl in (1, 2):
        s = s / _RES_DENOS[lvl]
        rows = [U_rows[i] - vals[i] for i in range(_N)]

        rank = []
        for i in range(_N):
            acc = None
            for j in range(_N):
                if j == i:
                    continue
                c = (rows[j] <= rows[i]) if j < i else (rows[j] < rows[i])
                ci = c.astype(jnp.int32)
                acc = ci if acc is None else acc + ci
            rank.append(acc)

        sorted_err = []
        for k in range(_N):
            acc = None
            for i in range(_N):
                t = jnp.where(rank[i] == k, rows[i], 0.0)
                acc = t if acc is None else acc + t
            sorted_err.append(acc)

        delta = [sorted_err[k + 1] - sorted_err[k] for k in range(_N - 1)]
        dmax = jnp.max(jnp.maximum(jnp.maximum(delta[0], delta[1]), delta[2]))
        delta = [d / dmax for d in delta]

        msp = [jax.nn.sigmoid((d - thres) / 0.01) for d in delta]
        rs = [(d > thres).astype(jnp.int32) for d in delta]

        code = rs[0] * 4 + rs[1] * 2 + rs[2]

        counts = [jnp.sum((code == c).astype(jnp.int32)) for c in range(_NCODES)]

        pos = []
        for i in range(_NCODES):
            acc = jnp.int32(0)
            for j in range(_NCODES):
                if j == i:
                    continue
                if j < i:
                    acc = acc + (counts[j] >= counts[i]).astype(jnp.int32)
                else:
                    acc = acc + (counts[j] > counts[i]).astype(jnp.int32)
            pos.append(acc)
        order = []
        for p in range(_NCODES):
            acc = jnp.int32(0)
            for i in range(_NCODES):
                acc = acc + jnp.where(pos[i] == p, jnp.int32(i), jnp.int32(0))
            order.append(acc)
        top = order[:_NUM_PART]

        clustered = code
        for j in range(_NCODES - _NUM_PART):
            less = order[_NUM_PART + j]
            lb = [(less // 4) % 2, (less // 2) % 2, less % 2]
            best_val = None
            winner = None
            for t in range(_NUM_PART):
                tb = [(top[t] // 4) % 2, (top[t] // 2) % 2, top[t] % 2]
                inter = lb[0] * tb[0] + lb[1] * tb[1] + lb[2] * tb[2]
                if best_val is None:
                    best_val, winner = inter, top[t]
                else:
                    better = inter > best_val
                    best_val = jnp.where(better, inter, best_val)
                    winner = jnp.where(better, top[t], winner)
            clustered = jnp.where(clustered == less, winner, clustered)

        running = jnp.int32(0)
        uidx = []
        for c in range(_NCODES):
            running = running + (counts[c] > 0).astype(jnp.int32)
            uidx.append(running - 1)
        sidx = None
        for c in range(_NCODES):
            t = jnp.where(clustered == c, uidx[c], jnp.int32(0))
            sidx = t if sidx is None else sidx + t
        sidx = jnp.clip(sidx, 0, _NCODES - 1)

        sel = None
        m = [None] * (_N - 1)
        for p in range(_NCODES):
            col_mask = (sub == 0) & (lane == p)
            code_p = jnp.sum(jnp.where(col_mask, code, jnp.int32(0)))
            hit = sidx == p
            t = jnp.where(hit, code_p, jnp.int32(0))
            sel = t if sel is None else sel + t
            for k in range(_N - 1):
                msp_p = jnp.sum(jnp.where(col_mask, msp[k], 0.0))
                tv = jnp.where(hit, msp_p, 0.0)
                m[k] = tv if m[k] is None else m[k] + tv

        t_rows = []
        for pattern in range(_NCODES):
            bits = [(pattern >> (_N - 2 - i)) & 1 for i in range(_N - 1)]
            buf = jnp.zeros((_D1, _D2), jnp.float32)
            cnt = 0
            grad_ = jnp.ones((_D1, _D2), jnp.float32)
            rows_out = []
            for i in range(_N):
                buf = buf + sorted_err[i]
                cnt += 1
                if i == _N - 1:
                    v = grad_ * (buf / cnt)
                    rows_out.extend([v] * cnt)
                elif bits[i]:
                    grad_ = grad_ * m[i]
                    v = grad_ * (buf / cnt)
                    rows_out.extend([v] * cnt)
                    buf = jnp.zeros((_D1, _D2), jnp.float32)
                    cnt = 0
                    grad_ = jnp.ones((_D1, _D2), jnp.float32)
                else:
                    grad_ = grad_ * (1.0 - m[i])
            t_rows.append(rows_out)

        inner = []
        for c in range(_N):
            acc = None
            for pattern in range(_NCODES):
                t = jnp.where(sel == pattern, t_rows[pattern][c], 0.0)
                acc = t if acc is None else acc + t
            inner.append(acc)
        for i in range(_N):
            acc = None
            for c in range(_N):
                t = jnp.where(rank[i] == c, inner[c], 0.0)
                acc = t if acc is None else acc + t
            vals[i] = vals[i] + s * _ste_floor(acc / s)

    return vals


def _fused_kernel(u_ref, tm_ref, x_ref, o_ref, wblk_ref):
    @pl.when(pl.program_id(0) == 0)
    def _quant_step():
        U_rows = [u_ref[i * _D1:(i + 1) * _D1, :] for i in range(_N)]
        tm = tm_ref[...]
        lane16 = jax.lax.broadcasted_iota(jnp.int32, (1, _D2), 1)
        sub = jax.lax.broadcasted_iota(jnp.int32, (_D1, _D2), 0)
        thres = jnp.zeros((_D1, _D2), jnp.float32)
        for g in range(_D2):
            tm_g = jnp.sum(jnp.where(lane16 == g, tm, 0.0))
            thres = thres + jnp.where((sub // 4) == g, tm_g, 0.0)
        thres = jax.nn.sigmoid(thres)

        vals = _quantize(U_rows, thres)

        wblk_ref[...] = jnp.zeros((_N * _D1, _N * _D2), jnp.float32)
        for n in range(_N):
            wblk_ref[n * _D1:(n + 1) * _D1, n * _D2:(n + 1) * _D2] = vals[n]

    o_ref[...] = jnp.dot(x_ref[...], wblk_ref[...],
                         preferred_element_type=jnp.float32)


def kernel(x, U, thres_mean, thres_var):
    del thres_var
    u2 = U.reshape(_N * _D1, _D2)
    tm = thres_mean.reshape(1, _D2)
    x2 = x.reshape(_B, _N * _D1)

    out = pl.pallas_call(
        _fused_kernel,
        grid=(_B // _BT,),
        in_specs=[
            pl.BlockSpec((_N * _D1, _D2), lambda i: (0, 0)),
            pl.BlockSpec((1, _D2), lambda i: (0, 0)),
            pl.BlockSpec((_BT, _N * _D1), lambda i: (i, 0)),
        ],
        out_specs=pl.BlockSpec((_BT, _N * _D2), lambda i: (i, 0)),
        out_shape=jax.ShapeDtypeStruct((_B, _N * _D2), jnp.float32),
        scratch_shapes=[pltpu.VMEM((_N * _D1, _N * _D2), jnp.float32)],
    )(u2, tm, x2)
    return out.reshape(_B, _N, _D2)


# speedup vs baseline: 2.2270x; 1.0213x over previous
import jax
import jax.numpy as jnp
from jax.experimental import pallas as pl
from jax.experimental.pallas import tpu as pltpu

_N = 4
_D1 = 64
_D2 = 16
_D = _D1 * _D2
_B = 32768
_NCODES = 8
_NUM_PART = 5
_RES_DENOS = (float(2**2 - 1), float(2**2 + 1), float(2**4 + 1))
_BT = 8192


def _ste_floor(x):
    return x + (jnp.floor(x) - x)


def _quantize(U_rows, thres):
    allU = jnp.concatenate(U_rows, axis=0)
    beta = jnp.max(allU)
    alpha = jnp.min(allU)

    sub = jax.lax.broadcasted_iota(jnp.int32, (_D1, _D2), 0)
    lane = jax.lax.broadcasted_iota(jnp.int32, (_D1, _D2), 1)

    s = (beta - alpha) / _RES_DENOS[0]
    vals = [s * _ste_floor(u / s) for u in U_rows]

    for lvl in (1, 2):
        s = s / _RES_DENOS[lvl]
        rows = [U_rows[i] - vals[i] for i in range(_N)]

        rank = []
        for i in range(_N):
            acc = None
            for j in range(_N):
                if j == i:
                    continue
                c = (rows[j] <= rows[i]) if j < i else (rows[j] < rows[i])
                ci = c.astype(jnp.int32)
                acc = ci if acc is None else acc + ci
            rank.append(acc)

        sorted_err = []
        for k in range(_N):
            acc = None
            for i in range(_N):
                t = jnp.where(rank[i] == k, rows[i], 0.0)
                acc = t if acc is None else acc + t
            sorted_err.append(acc)

        delta = [sorted_err[k + 1] - sorted_err[k] for k in range(_N - 1)]
        dmax = jnp.max(jnp.maximum(jnp.maximum(delta[0], delta[1]), delta[2]))
        delta = [d / dmax for d in delta]

        msp = [jax.nn.sigmoid((d - thres) / 0.01) for d in delta]
        rs = [(d > thres).astype(jnp.int32) for d in delta]

        code = rs[0] * 4 + rs[1] * 2 + rs[2]

        counts = [jnp.sum((code == c).astype(jnp.int32)) for c in range(_NCODES)]

        pos = []
        for i in range(_NCODES):
            acc = jnp.int32(0)
            for j in range(_NCODES):
                if j == i:
                    continue
                if j < i:
                    acc = acc + (counts[j] >= counts[i]).astype(jnp.int32)
                else:
                    acc = acc + (counts[j] > counts[i]).astype(jnp.int32)
            pos.append(acc)
        order = []
        for p in range(_NCODES):
            acc = jnp.int32(0)
            for i in range(_NCODES):
                acc = acc + jnp.where(pos[i] == p, jnp.int32(i), jnp.int32(0))
            order.append(acc)
        top = order[:_NUM_PART]

        clustered = code
        for j in range(_NCODES - _NUM_PART):
            less = order[_NUM_PART + j]
            lb = [(less // 4) % 2, (less // 2) % 2, less % 2]
            best_val = None
            winner = None
            for t in range(_NUM_PART):
                tb = [(top[t] // 4) % 2, (top[t] // 2) % 2, top[t] % 2]
                inter = lb[0] * tb[0] + lb[1] * tb[1] + lb[2] * tb[2]
                if best_val is None:
                    best_val, winner = inter, top[t]
                else:
                    better = inter > best_val
                    best_val = jnp.where(better, inter, best_val)
                    winner = jnp.where(better, top[t], winner)
            clustered = jnp.where(clustered == less, winner, clustered)

        running = jnp.int32(0)
        uidx = []
        for c in range(_NCODES):
            running = running + (counts[c] > 0).astype(jnp.int32)
            uidx.append(running - 1)
        sidx = None
        for c in range(_NCODES):
            t = jnp.where(clustered == c, uidx[c], jnp.int32(0))
            sidx = t if sidx is None else sidx + t
        sidx = jnp.clip(sidx, 0, _NCODES - 1)

        sel = None
        m = [None] * (_N - 1)
        for p in range(_NCODES):
            col_mask = (sub == 0) & (lane == p)
            code_p = jnp.sum(jnp.where(col_mask, code, jnp.int32(0)))
            hit = sidx == p
            t = jnp.where(hit, code_p, jnp.int32(0))
            sel = t if sel is None else sel + t
            for k in range(_N - 1):
                msp_p = jnp.sum(jnp.where(col_mask, msp[k], 0.0))
                tv = jnp.where(hit, msp_p, 0.0)
                m[k] = tv if m[k] is None else m[k] + tv

        t_rows = []
        for pattern in range(_NCODES):
            bits = [(pattern >> (_N - 2 - i)) & 1 for i in range(_N - 1)]
            buf = jnp.zeros((_D1, _D2), jnp.float32)
            cnt = 0
            grad_ = jnp.ones((_D1, _D2), jnp.float32)
            rows_out = []
            for i in range(_N):
                buf = buf + sorted_err[i]
                cnt += 1
                if i == _N - 1:
                    v = grad_ * (buf / cnt)
                    rows_out.extend([v] * cnt)
                elif bits[i]:
                    grad_ = grad_ * m[i]
                    v = grad_ * (buf / cnt)
                    rows_out.extend([v] * cnt)
                    buf = jnp.zeros((_D1, _D2), jnp.float32)
                    cnt = 0
                    grad_ = jnp.ones((_D1, _D2), jnp.float32)
                else:
                    grad_ = grad_ * (1.0 - m[i])
            t_rows.append(rows_out)

        inner = []
        for c in range(_N):
            acc = None
            for pattern in range(_NCODES):
                t = jnp.where(sel == pattern, t_rows[pattern][c], 0.0)
                acc = t if acc is None else acc + t
            inner.append(acc)
        for i in range(_N):
            acc = None
            for c in range(_N):
                t = jnp.where(rank[i] == c, inner[c], 0.0)
                acc = t if acc is None else acc + t
            vals[i] = vals[i] + s * _ste_floor(acc / s)

    return vals


def _fused_kernel(u_ref, tm_ref, x_ref, o_ref, wblk_ref):
    @pl.when(pl.program_id(0) == 0)
    def _quant_step():
        U_rows = [u_ref[i * _D1:(i + 1) * _D1, :] for i in range(_N)]
        tm = tm_ref[...]
        lane16 = jax.lax.broadcasted_iota(jnp.int32, (1, _D2), 1)
        sub = jax.lax.broadcasted_iota(jnp.int32, (_D1, _D2), 0)
        thres = jnp.zeros((_D1, _D2), jnp.float32)
        for g in range(_D2):
            tm_g = jnp.sum(jnp.where(lane16 == g, tm, 0.0))
            thres = thres + jnp.where((sub // 4) == g, tm_g, 0.0)
        thres = jax.nn.sigmoid(thres)

        vals = _quantize(U_rows, thres)

        wblk_ref[...] = jnp.zeros((_N * _D1, _N * _D2), jnp.float32)
        for n in range(_N):
            wblk_ref[n * _D1:(n + 1) * _D1, n * _D2:(n + 1) * _D2] = vals[n]

    o_ref[...] = jnp.dot(x_ref[...], wblk_ref[...],
                         preferred_element_type=jnp.float32)


def kernel(x, U, thres_mean, thres_var):
    del thres_var
    u2 = U.reshape(_N * _D1, _D2)
    tm = thres_mean.reshape(1, _D2)
    x2 = x.reshape(_B, _N * _D1)

    out = pl.pallas_call(
        _fused_kernel,
        grid=(_B // _BT,),
        in_specs=[
            pl.BlockSpec((_N * _D1, _D2), lambda i: (0, 0)),
            pl.BlockSpec((1, _D2), lambda i: (0, 0)),
            pl.BlockSpec((_BT, _N * _D1), lambda i: (i, 0)),
        ],
        out_specs=pl.BlockSpec((_BT, _N * _D2), lambda i: (i, 0)),
        out_shape=jax.ShapeDtypeStruct((_B, _N * _D2), jnp.float32),
        scratch_shapes=[pltpu.VMEM((_N * _D1, _N * _D2), jnp.float32)],
    )(u2, tm, x2)
    return out.reshape(_B, _N, _D2)


# speedup vs baseline: 2.3390x; 1.0503x over previous
import jax
import jax.numpy as jnp
from jax.experimental import pallas as pl
from jax.experimental.pallas import tpu as pltpu

_N = 4
_D1 = 64
_D2 = 16
_D = _D1 * _D2
_B = 32768
_NCODES = 8
_NUM_PART = 5
_RES_DENOS = (float(2**2 - 1), float(2**2 + 1), float(2**4 + 1))
_BT = 16384


def _ste_floor(x):
    return x + (jnp.floor(x) - x)


def _quantize(U_rows, thres):
    allU = jnp.concatenate(U_rows, axis=0)
    beta = jnp.max(allU)
    alpha = jnp.min(allU)

    sub = jax.lax.broadcasted_iota(jnp.int32, (_D1, _D2), 0)
    lane = jax.lax.broadcasted_iota(jnp.int32, (_D1, _D2), 1)

    s = (beta - alpha) / _RES_DENOS[0]
    vals = [s * _ste_floor(u / s) for u in U_rows]

    for lvl in (1, 2):
        s = s / _RES_DENOS[lvl]
        rows = [U_rows[i] - vals[i] for i in range(_N)]

        rank = []
        for i in range(_N):
            acc = None
            for j in range(_N):
                if j == i:
                    continue
                c = (rows[j] <= rows[i]) if j < i else (rows[j] < rows[i])
                ci = c.astype(jnp.int32)
                acc = ci if acc is None else acc + ci
            rank.append(acc)

        sorted_err = []
        for k in range(_N):
            acc = None
            for i in range(_N):
                t = jnp.where(rank[i] == k, rows[i], 0.0)
                acc = t if acc is None else acc + t
            sorted_err.append(acc)

        delta = [sorted_err[k + 1] - sorted_err[k] for k in range(_N - 1)]
        dmax = jnp.max(jnp.maximum(jnp.maximum(delta[0], delta[1]), delta[2]))
        delta = [d / dmax for d in delta]

        msp = [jax.nn.sigmoid((d - thres) / 0.01) for d in delta]
        rs = [(d > thres).astype(jnp.int32) for d in delta]

        code = rs[0] * 4 + rs[1] * 2 + rs[2]

        counts = [jnp.sum((code == c).astype(jnp.int32)) for c in range(_NCODES)]

        pos = []
        for i in range(_NCODES):
            acc = jnp.int32(0)
            for j in range(_NCODES):
                if j == i:
                    continue
                if j < i:
                    acc = acc + (counts[j] >= counts[i]).astype(jnp.int32)
                else:
                    acc = acc + (counts[j] > counts[i]).astype(jnp.int32)
            pos.append(acc)
        order = []
        for p in range(_NCODES):
            acc = jnp.int32(0)
            for i in range(_NCODES):
                acc = acc + jnp.where(pos[i] == p, jnp.int32(i), jnp.int32(0))
            order.append(acc)
        top = order[:_NUM_PART]

        clustered = code
        for j in range(_NCODES - _NUM_PART):
            less = order[_NUM_PART + j]
            lb = [(less // 4) % 2, (less // 2) % 2, less % 2]
            best_val = None
            winner = None
            for t in range(_NUM_PART):
                tb = [(top[t] // 4) % 2, (top[t] // 2) % 2, top[t] % 2]
                inter = lb[0] * tb[0] + lb[1] * tb[1] + lb[2] * tb[2]
                if best_val is None:
                    best_val, winner = inter, top[t]
                else:
                    better = inter > best_val
                    best_val = jnp.where(better, inter, best_val)
                    winner = jnp.where(better, top[t], winner)
            clustered = jnp.where(clustered == less, winner, clustered)

        running = jnp.int32(0)
        uidx = []
        for c in range(_NCODES):
            running = running + (counts[c] > 0).astype(jnp.int32)
            uidx.append(running - 1)
        sidx = None
        for c in range(_NCODES):
            t = jnp.where(clustered == c, uidx[c], jnp.int32(0))
            sidx = t if sidx is None else sidx + t
        sidx = jnp.clip(sidx, 0, _NCODES - 1)

        sel = None
        m = [None] * (_N - 1)
        for p in range(_NCODES):
            col_mask = (sub == 0) & (lane == p)
            code_p = jnp.sum(jnp.where(col_mask, code, jnp.int32(0)))
            hit = sidx == p
            t = jnp.where(hit, code_p, jnp.int32(0))
            sel = t if sel is None else sel + t
            for k in range(_N - 1):
                msp_p = jnp.sum(jnp.where(col_mask, msp[k], 0.0))
                tv = jnp.where(hit, msp_p, 0.0)
                m[k] = tv if m[k] is None else m[k] + tv

        t_rows = []
        for pattern in range(_NCODES):
            bits = [(pattern >> (_N - 2 - i)) & 1 for i in range(_N - 1)]
            buf = jnp.zeros((_D1, _D2), jnp.float32)
            cnt = 0
            grad_ = jnp.ones((_D1, _D2), jnp.float32)
            rows_out = []
            for i in range(_N):
                buf = buf + sorted_err[i]
                cnt += 1
                if i == _N - 1:
                    v = grad_ * (buf / cnt)
                    rows_out.extend([v] * cnt)
                elif bits[i]:
                    grad_ = grad_ * m[i]
                    v = grad_ * (buf / cnt)
                    rows_out.extend([v] * cnt)
                    buf = jnp.zeros((_D1, _D2), jnp.float32)
                    cnt = 0
                    grad_ = jnp.ones((_D1, _D2), jnp.float32)
                else:
                    grad_ = grad_ * (1.0 - m[i])
            t_rows.append(rows_out)

        inner = []
        for c in range(_N):
            acc = None
            for pattern in range(_NCODES):
                t = jnp.where(sel == pattern, t_rows[pattern][c], 0.0)
                acc = t if acc is None else acc + t
            inner.append(acc)
        for i in range(_N):
            acc = None
            for c in range(_N):
                t = jnp.where(rank[i] == c, inner[c], 0.0)
                acc = t if acc is None else acc + t
            vals[i] = vals[i] + s * _ste_floor(acc / s)

    return vals


def _fused_kernel(u_ref, tm_ref, x_ref, o_ref, wblk_ref):
    @pl.when(pl.program_id(0) == 0)
    def _quant_step():
        U_rows = [u_ref[i * _D1:(i + 1) * _D1, :] for i in range(_N)]
        tm = tm_ref[...]
        lane16 = jax.lax.broadcasted_iota(jnp.int32, (1, _D2), 1)
        sub = jax.lax.broadcasted_iota(jnp.int32, (_D1, _D2), 0)
        thres = jnp.zeros((_D1, _D2), jnp.float32)
        for g in range(_D2):
            tm_g = jnp.sum(jnp.where(lane16 == g, tm, 0.0))
            thres = thres + jnp.where((sub // 4) == g, tm_g, 0.0)
        thres = jax.nn.sigmoid(thres)

        vals = _quantize(U_rows, thres)

        wblk_ref[...] = jnp.zeros((_N * _D1, _N * _D2), jnp.float32)
        for n in range(_N):
            wblk_ref[n * _D1:(n + 1) * _D1, n * _D2:(n + 1) * _D2] = vals[n]

    o_ref[...] = jnp.dot(x_ref[...], wblk_ref[...],
                         preferred_element_type=jnp.float32)


def kernel(x, U, thres_mean, thres_var):
    del thres_var
    u2 = U.reshape(_N * _D1, _D2)
    tm = thres_mean.reshape(1, _D2)
    x2 = x.reshape(_B, _N * _D1)

    out = pl.pallas_call(
        _fused_kernel,
        grid=(_B // _BT,),
        in_specs=[
            pl.BlockSpec((_N * _D1, _D2), lambda i: (0, 0)),
            pl.BlockSpec((1, _D2), lambda i: (0, 0)),
            pl.BlockSpec((_BT, _N * _D1), lambda i: (i, 0)),
        ],
        out_specs=pl.BlockSpec((_BT, _N * _D2), lambda i: (i, 0)),
        out_shape=jax.ShapeDtypeStruct((_B, _N * _D2), jnp.float32),
        scratch_shapes=[pltpu.VMEM((_N * _D1, _N * _D2), jnp.float32)],
    )(u2, tm, x2)
    return out.reshape(_B, _N, _D2)
